# Initial kernel scaffold; baseline (speedup 1.0000x reference)
import jax
import jax.numpy as jnp
from jax.experimental import pallas as pl


def kernel(x, y, x_aux, y_aux, lamb_hat, indices, centers):
    raise NotImplementedError("write your pallas kernel here")


# speedup vs baseline: 1.3439x; 1.3439x over previous
import functools

import jax
import jax.numpy as jnp
from jax import lax
from jax.experimental import pallas as pl
from jax.experimental.pallas import tpu as pltpu
from jax.experimental.pallas import tpu_sc as plsc

B, K, D = 4096, 20, 64
N = B * (1 + K)
R = 128
NCHUNK = N // R
NW = 32
CPW = NCHUNK // NW
_RD = R * D


def _sc_body(xf, auxf, yy3, li3, lamb, centers, out,
             yyb, lib, xb0, xb1, cb0, cb1, lb0, lb1, accv,
             sx0, sx1, sc0, sc1, sl0, sl1):
    w = lax.axis_index("s") * 2 + lax.axis_index("c")

    slot0 = (xb0, cb0, lb0, sx0, sc0, sl0)
    slot1 = (xb1, cb1, lb1, sx1, sc1, sl1)

    pltpu.sync_copy(yy3.at[:, w], yyb)
    pltpu.sync_copy(li3.at[:, w], lib)

    def fire(c, xsrc, xoff, slot):
        xb, cb, lb, sx, sc, sl = slot
        pltpu.async_copy(xsrc.at[pl.ds(xoff, _RD)], xb, sx)
        pltpu.async_copy(centers.at[yyb.at[c]], cb, sc)
        pltpu.async_copy(lamb.at[lib.at[c]], lb, sl)

    def wait(slot):
        xb, cb, lb, sx, sc, sl = slot
        pltpu.make_async_copy(xf.at[pl.ds(0, _RD)], xb, sx).wait()
        pltpu.make_async_copy(centers.at[yyb.at[0]], cb, sc).wait()
        pltpu.make_async_copy(lamb.at[lib.at[0]], lb, sl).wait()

    def aux_off(c):
        return (w + 32 * c - 32) * _RD

    def compute(acc, slot):
        xb, cb, lb, _, _, _ = slot

        def gbody(g, acc):
            lam16 = 1.0 / (1.0 + jnp.exp(-lb[pl.ds(g * 16, 16)]))
            for i in range(16):
                lam = jnp.broadcast_to(lam16[i], (16,))
                r = g * 16 + i
                for kk in range(4):
                    xv = xb[pl.ds(r * D + kk * 16, 16)]
                    cv = cb[r, pl.ds(kk * 16, 16)]
                    dv = xv - cv
                    acc = acc + lam * (dv * dv)
            return acc

        return lax.fori_loop(0, R // 16, gbody, acc)

    fire(0, xf, w * _RD, slot0)
    fire(1, auxf, aux_off(1), slot1)

    acc = jnp.zeros((16,), jnp.float32)
    wait(slot0)
    acc = compute(acc, slot0)
    fire(2, auxf, aux_off(2), slot0)

    def jbody(j, acc):
        c1 = 2 * j + 1
        wait(slot1)
        acc = compute(acc, slot1)
        fire(c1 + 2, auxf, aux_off(c1 + 2), slot1)
        c0 = 2 * j + 2
        wait(slot0)
        acc = compute(acc, slot0)
        fire(c0 + 2, auxf, aux_off(c0 + 2), slot0)
        return acc

    acc = lax.fori_loop(0, (CPW - 3) // 2, jbody, acc)

    wait(slot1)
    acc = compute(acc, slot1)
    wait(slot0)
    acc = compute(acc, slot0)

    accv[...] = acc
    pltpu.sync_copy(accv, out.at[w])


_disc_loss_sc = functools.partial(
    pl.kernel,
    mesh=plsc.VectorSubcoreMesh(core_axis_name="c", subcore_axis_name="s"),
    out_type=jax.ShapeDtypeStruct((NW, 16), jnp.float32),
    scratch_types=[
        pltpu.VMEM((CPW, R), jnp.int32),
        pltpu.VMEM((CPW, R), jnp.int32),
        pltpu.VMEM((_RD,), jnp.float32),
        pltpu.VMEM((_RD,), jnp.float32),
        pltpu.VMEM((R, D), jnp.float32),
        pltpu.VMEM((R, D), jnp.float32),
        pltpu.VMEM((R,), jnp.float32),
        pltpu.VMEM((R,), jnp.float32),
        pltpu.VMEM((16,), jnp.float32),
        pltpu.SemaphoreType.DMA,
        pltpu.SemaphoreType.DMA,
        pltpu.SemaphoreType.DMA,
        pltpu.SemaphoreType.DMA,
        pltpu.SemaphoreType.DMA,
        pltpu.SemaphoreType.DMA,
    ],
    compiler_params=pltpu.CompilerParams(use_tc_tiling_on_sc=False),
)(_sc_body)


def kernel(x, y, x_aux, y_aux, lamb_hat, indices, centers):
    xf = x.reshape(-1)
    auxf = x_aux.reshape(-1)
    yy = jnp.concatenate([y, y_aux.reshape(-1)]).astype(jnp.int32)
    li = indices.reshape(-1).astype(jnp.int32)
    yy3 = yy.reshape(CPW, NW, R)
    li3 = li.reshape(CPW, NW, R)
    out = _disc_loss_sc(xf, auxf, yy3, li3, lamb_hat, centers)
    return 0.5 * jnp.sum(out) / jnp.float32(N)


# speedup vs baseline: 2.3661x; 1.7607x over previous
import functools

import jax
import jax.numpy as jnp
from jax import lax
from jax.experimental import pallas as pl
from jax.experimental.pallas import tpu as pltpu
from jax.experimental.pallas import tpu_sc as plsc

B, K, D = 4096, 20, 64
N = B * (1 + K)
NW = 32
NSEG = K + 1
EPT = N // 16
NC = EPT // 128
QS = 32767.0
MASK17 = (1 << 17) - 1


def _sc_body(xT, xauxT, yyp, lip, lamb, cT, out, pk,
             tbl, a0, a1, f0, f1, accv,
             s_tbl, s_g, s_x0, s_x1, s_p0, s_p1):
    cid = lax.axis_index("c")
    sid = lax.axis_index("s")
    wid = sid * 2 + cid

    d0 = wid * 2
    pltpu.async_copy(cT.at[d0], tbl, s_tbl)
    pltpu.async_copy(xauxT.at[1, d0], f1.at[pl.ds(0, B)], s_x1)

    base = sid * EPT
    pkoff = cid * N
    pltpu.sync_copy(lip.at[pl.ds(base, EPT)], a0)
    pltpu.sync_copy(yyp.at[pl.ds(base, EPT)], a1)
    cps = [
        pltpu.async_copy(lamb.at[a0.at[pl.ds(j * 128, 128)]],
                         f0.at[pl.ds(j * 128, 128)], s_g)
        for j in range(NC)
    ]
    for j in range(NC):
        cps[j].wait()
        for i in range(j * 8, j * 8 + 8):
            sig = 1.0 / (1.0 + jnp.exp(-f0[pl.ds(i * 16, 16)]))
            q = (sig * QS + 0.5).astype(jnp.int32)
            a0[pl.ds(i * 16, 16)] = a1[pl.ds(i * 16, 16)] | (q << 17)
    pltpu.sync_copy(a0, pk.at[pl.ds(pkoff + base, EPT)])
    plsc.subcore_barrier()

    def compute(acc, xb, pb):
        def ubody(j, accs):
            accs = list(accs)
            for u in range(8):
                i = j * 8 + u
                w = pb[pl.ds(i * 16, 16)]
                idx = w & MASK17
                q = lax.shift_right_logical(w, 17)
                c16 = plsc.load_gather(tbl, [idx])
                x16 = xb[pl.ds(i * 16, 16)]
                dv = x16 - c16
                accs[u % 4] = accs[u % 4] + q.astype(jnp.float32) * (dv * dv)
            return tuple(accs)

        z = jnp.zeros((16,), jnp.float32)
        a, b, c, d = lax.fori_loop(0, B // 128, ubody, (acc, z, z, z))
        return a + (b + (c + d))

    slot = ((f0, a0, s_x0, s_p0), (f1, a1, s_x1, s_p1))

    def fire_aux(seg, dd, sl):
        xb, pb, sx, sp = sl
        pltpu.async_copy(xauxT.at[seg, dd], xb.at[pl.ds(0, B)], sx)
        pltpu.async_copy(pk.at[pl.ds(pkoff + seg * B, B)], pb.at[pl.ds(0, B)], sp)

    def fire_x(dd, sl):
        xb, pb, sx, sp = sl
        pltpu.async_copy(xT.at[dd], xb.at[pl.ds(0, B)], sx)
        pltpu.async_copy(pk.at[pl.ds(pkoff + K * B, B)], pb.at[pl.ds(0, B)], sp)

    def wait_slot(sl):
        xb, pb, sx, sp = sl
        pltpu.make_async_copy(xT.at[0], xb.at[pl.ds(0, B)], sx).wait()
        pltpu.make_async_copy(yyp.at[pl.ds(0, B)], pb.at[pl.ds(0, B)], sp).wait()

    def make_jbody(dd):
        def jbody(j, acc):
            acc_ = acc
            for h in range(2):
                sl = slot[h]
                seg = 2 * j + h
                wait_slot(sl)
                acc_ = compute(acc_, sl[0], sl[1])
                fire_aux(seg + 2, dd, sl)
            return acc_
        return jbody

    acc = jnp.zeros((16,), jnp.float32)

    fire_aux(0, d0, slot[0])
    pltpu.async_copy(pk.at[pl.ds(pkoff + B, B)], a1.at[pl.ds(0, B)], s_p1)
    pltpu.make_async_copy(cT.at[0], tbl, s_tbl).wait()
    acc = lax.fori_loop(0, (K - 2) // 2, make_jbody(d0), acc)
    wait_slot(slot[0])
    acc = compute(acc, slot[0][0], slot[0][1])
    fire_x(d0, slot[0])
    wait_slot(slot[1])
    acc = compute(acc, slot[1][0], slot[1][1])
    fire_aux(1, d0 + 1, slot[1])
    wait_slot(slot[0])
    acc = compute(acc, slot[0][0], slot[0][1])
    pltpu.async_copy(cT.at[d0 + 1], tbl, s_tbl)
    fire_aux(0, d0 + 1, slot[0])

    pltpu.make_async_copy(cT.at[0], tbl, s_tbl).wait()
    acc = lax.fori_loop(0, (K - 2) // 2, make_jbody(d0 + 1), acc)
    wait_slot(slot[0])
    acc = compute(acc, slot[0][0], slot[0][1])
    fire_x(d0 + 1, slot[0])
    wait_slot(slot[1])
    acc = compute(acc, slot[1][0], slot[1][1])
    wait_slot(slot[0])
    acc = compute(acc, slot[0][0], slot[0][1])

    accv[...] = acc * (1.0 / QS)
    pltpu.sync_copy(accv, out.at[pl.ds(wid * 16, 16)])


_disc_loss_sc = functools.partial(
    pl.kernel,
    mesh=plsc.VectorSubcoreMesh(core_axis_name="c", subcore_axis_name="s"),
    out_type=(
        jax.ShapeDtypeStruct((NW * 16,), jnp.float32),
        jax.ShapeDtypeStruct((2 * N,), jnp.int32),
    ),
    scratch_types=[
        pltpu.VMEM((100000,), jnp.float32),
        pltpu.VMEM((EPT,), jnp.int32),
        pltpu.VMEM((EPT,), jnp.int32),
        pltpu.VMEM((EPT,), jnp.float32),
        pltpu.VMEM((B,), jnp.float32),
        pltpu.VMEM((16,), jnp.float32),
        pltpu.SemaphoreType.DMA,
        pltpu.SemaphoreType.DMA,
        pltpu.SemaphoreType.DMA,
        pltpu.SemaphoreType.DMA,
        pltpu.SemaphoreType.DMA,
        pltpu.SemaphoreType.DMA,
    ],
    compiler_params=pltpu.CompilerParams(
        use_tc_tiling_on_sc=True, needs_layout_passes=False
    ),
)(_sc_body)


def kernel(x, y, x_aux, y_aux, lamb_hat, indices, centers):
    xT = x.T
    xauxT = jnp.transpose(x_aux, (1, 2, 0))
    cT = centers.T
    yyp = jnp.concatenate([y_aux.T.reshape(-1), y]).astype(jnp.int32)
    li = indices.reshape(-1).astype(jnp.int32)
    lip = jnp.concatenate([li[B:].reshape(B, K).T.reshape(-1), li[:B]])
    partials, _ = _disc_loss_sc(xT, xauxT, yyp, lip, lamb_hat, cT)
    return 0.5 * jnp.sum(partials) / jnp.float32(N)


# speedup vs baseline: 2.3935x; 1.0116x over previous
import functools

import jax
import jax.numpy as jnp
from jax import lax
from jax.experimental import pallas as pl
from jax.experimental.pallas import tpu as pltpu
from jax.experimental.pallas import tpu_sc as plsc

B, K, D = 4096, 20, 64
N = B * (1 + K)
NW = 32
NSEG = K + 1
EPT = N // 16
NC = EPT // 128
QS = 32767.0
MASK17 = (1 << 17) - 1


def _sc_body(xT, xauxT, yyp, lip, lamb, cT, out, pk,
             tbl, a0, a1, f0, f1, accv,
             s_tbl, s_g, s_x0, s_x1, s_p0, s_p1):
    cid = lax.axis_index("c")
    sid = lax.axis_index("s")
    wid = sid * 2 + cid

    d0 = wid * 2
    pltpu.async_copy(cT.at[d0], tbl, s_tbl)
    pltpu.async_copy(xauxT.at[1, d0], f1.at[pl.ds(0, B)], s_x1)

    base = sid * EPT
    pkoff = cid * N
    pltpu.sync_copy(lip.at[pl.ds(base, EPT)], a0)
    pltpu.sync_copy(yyp.at[pl.ds(base, EPT)], a1)
    cps = [
        pltpu.async_copy(lamb.at[a0.at[pl.ds(j * 128, 128)]],
                         f0.at[pl.ds(j * 128, 128)], s_g)
        for j in range(NC)
    ]
    for c in cps:
        c.wait()
    for i in range(EPT // 16):
        sig = 1.0 / (1.0 + jnp.exp(-f0[pl.ds(i * 16, 16)]))
        q = (sig * QS + 0.5).astype(jnp.int32)
        a0[pl.ds(i * 16, 16)] = a1[pl.ds(i * 16, 16)] | (q << 17)
    pltpu.sync_copy(a0, pk.at[pl.ds(pkoff + base, EPT)])
    plsc.subcore_barrier()

    def compute(acc, xb, pb):
        def ubody(j, accs):
            accs = list(accs)
            for u in range(8):
                i = j * 8 + u
                w = pb[pl.ds(i * 16, 16)]
                idx = w & MASK17
                q = lax.shift_right_logical(w, 17)
                c16 = plsc.load_gather(tbl, [idx])
                x16 = xb[pl.ds(i * 16, 16)]
                dv = x16 - c16
                accs[u % 4] = accs[u % 4] + q.astype(jnp.float32) * (dv * dv)
            return tuple(accs)

        z = jnp.zeros((16,), jnp.float32)
        a, b, c, d = lax.fori_loop(0, B // 128, ubody, (acc, z, z, z))
        return a + (b + (c + d))

    slot = ((f0, a0, s_x0, s_p0), (f1, a1, s_x1, s_p1))

    def fire_aux(seg, dd, sl):
        xb, pb, sx, sp = sl
        pltpu.async_copy(xauxT.at[seg, dd], xb.at[pl.ds(0, B)], sx)
        pltpu.async_copy(pk.at[pl.ds(pkoff + seg * B, B)], pb.at[pl.ds(0, B)], sp)

    def fire_x(dd, sl):
        xb, pb, sx, sp = sl
        pltpu.async_copy(xT.at[dd], xb.at[pl.ds(0, B)], sx)
        pltpu.async_copy(pk.at[pl.ds(pkoff + K * B, B)], pb.at[pl.ds(0, B)], sp)

    def wait_slot(sl):
        xb, pb, sx, sp = sl
        pltpu.make_async_copy(xT.at[0], xb.at[pl.ds(0, B)], sx).wait()
        pltpu.make_async_copy(yyp.at[pl.ds(0, B)], pb.at[pl.ds(0, B)], sp).wait()

    def make_jbody(dd):
        def jbody(j, acc):
            acc_ = acc
            for h in range(2):
                sl = slot[h]
                seg = 2 * j + h
                wait_slot(sl)
                acc_ = compute(acc_, sl[0], sl[1])
                fire_aux(seg + 2, dd, sl)
            return acc_
        return jbody

    acc = jnp.zeros((16,), jnp.float32)

    fire_aux(0, d0, slot[0])
    pltpu.async_copy(pk.at[pl.ds(pkoff + B, B)], a1.at[pl.ds(0, B)], s_p1)
    pltpu.make_async_copy(cT.at[0], tbl, s_tbl).wait()
    acc = lax.fori_loop(0, (K - 2) // 2, make_jbody(d0), acc)
    wait_slot(slot[0])
    acc = compute(acc, slot[0][0], slot[0][1])
    fire_x(d0, slot[0])
    wait_slot(slot[1])
    acc = compute(acc, slot[1][0], slot[1][1])
    fire_aux(1, d0 + 1, slot[1])
    wait_slot(slot[0])
    acc = compute(acc, slot[0][0], slot[0][1])
    pltpu.async_copy(cT.at[d0 + 1], tbl, s_tbl)
    fire_aux(0, d0 + 1, slot[0])

    pltpu.make_async_copy(cT.at[0], tbl, s_tbl).wait()
    acc = lax.fori_loop(0, (K - 2) // 2, make_jbody(d0 + 1), acc)
    wait_slot(slot[0])
    acc = compute(acc, slot[0][0], slot[0][1])
    fire_x(d0 + 1, slot[0])
    wait_slot(slot[1])
    acc = compute(acc, slot[1][0], slot[1][1])
    wait_slot(slot[0])
    acc = compute(acc, slot[0][0], slot[0][1])

    accv[...] = acc * (1.0 / QS)
    pltpu.sync_copy(accv, out.at[pl.ds(wid * 16, 16)])


_disc_loss_sc = functools.partial(
    pl.kernel,
    mesh=plsc.VectorSubcoreMesh(core_axis_name="c", subcore_axis_name="s"),
    out_type=(
        jax.ShapeDtypeStruct((NW * 16,), jnp.float32),
        jax.ShapeDtypeStruct((2 * N,), jnp.int32),
    ),
    scratch_types=[
        pltpu.VMEM((100000,), jnp.float32),
        pltpu.VMEM((EPT,), jnp.int32),
        pltpu.VMEM((EPT,), jnp.int32),
        pltpu.VMEM((EPT,), jnp.float32),
        pltpu.VMEM((B,), jnp.float32),
        pltpu.VMEM((16,), jnp.float32),
        pltpu.SemaphoreType.DMA,
        pltpu.SemaphoreType.DMA,
        pltpu.SemaphoreType.DMA,
        pltpu.SemaphoreType.DMA,
        pltpu.SemaphoreType.DMA,
        pltpu.SemaphoreType.DMA,
    ],
    compiler_params=pltpu.CompilerParams(
        use_tc_tiling_on_sc=True, needs_layout_passes=False
    ),
)(_sc_body)


def kernel(x, y, x_aux, y_aux, lamb_hat, indices, centers):
    xT = x.T
    xauxT = jnp.transpose(x_aux, (1, 2, 0))
    cT = centers.T
    yyp = jnp.concatenate([y_aux.T.reshape(-1), y]).astype(jnp.int32)
    li = indices.reshape(-1).astype(jnp.int32)
    lip = jnp.concatenate([li[B:].reshape(B, K).T.reshape(-1), li[:B]])
    partials, _ = _disc_loss_sc(xT, xauxT, yyp, lip, lamb_hat, cT)
    return 0.5 * jnp.sum(partials) / jnp.float32(N)
